# baseline (speedup 1.0000x reference)
import functools

import jax
import jax.numpy as jnp
from jax import lax
from jax.experimental import pallas as pl
from jax.experimental.pallas import tpu as pltpu
from jax.experimental.pallas import tpu_sc as plsc

B, S, D, E, TOP_K = 4, 4096, 4096, 64, 8
T = B * S

BT = 1024
CHUNK_TOKENS = (12288, 4096)

NC, NS, L = 2, 16, 16
NW = NC * NS


def _logits_body(x_ref, m_ref, w_ref, b_ref, lg_ref):
    lg = lax.dot_general(
        w_ref[...], x_ref[...],
        dimension_numbers=(((1,), (1,)), ((), ())),
        preferred_element_type=jnp.float32,
    )
    lg = lg + b_ref[...]
    lg_ref[...] = jnp.where(m_ref[...] != 1, -jnp.inf, lg)


_GATHER_DNUMS = lax.GatherDimensionNumbers(
    offset_dims=(), collapsed_slice_dims=(0,), start_index_map=(0,)
)


def _permute(v, idx):
    return lax.gather(
        v, idx[:, None], _GATHER_DNUMS, (1,),
        mode=lax.GatherScatterMode.PROMISE_IN_BOUNDS,
    )


def _make_route(tc):
    tok_w = tc // NW
    ng = tok_w // L

    def _route_body(lg_hbm, ew_hbm, ei_hbm, lg_v, ew_v, ei_v, sem):
        wid = lax.axis_index("s") * NC + lax.axis_index("c")
        base = wid * tok_w
        pltpu.sync_copy(lg_hbm.at[:, pl.ds(base, tok_w)], lg_v)

        neg_inf = jnp.full((L,), -jnp.inf, jnp.float32)
        lane = lax.iota(jnp.int32, L)
        lane_rank = jnp.bitwise_and(lane, 7)
        lane_tok = jnp.right_shift(lane, 3)
        rank_masks = [lane_rank == j for j in range(TOP_K)]

        def group(g, _):
            def expert(e, carry):
                topv = list(carry[:TOP_K])
                topi = list(carry[TOP_K:])
                xv = lg_v[e, pl.ds(g * L, L)]
                xi = jnp.full((L,), e, jnp.int32)
                for j in range(TOP_K):
                    c = xv > topv[j]
                    nv = jnp.where(c, xv, topv[j])
                    xv = jnp.where(c, topv[j], xv)
                    ni = jnp.where(c, xi, topi[j])
                    xi = jnp.where(c, topi[j], xi)
                    topv[j] = nv
                    topi[j] = ni
                return tuple(topv) + tuple(topi)

            init = (neg_inf,) * TOP_K + (jnp.zeros((L,), jnp.int32),) * TOP_K
            carry = lax.fori_loop(0, E, expert, init)
            topv = list(carry[:TOP_K])
            topi = list(carry[TOP_K:])
            es = [jnp.exp(v - topv[0]) for v in topv]
            tot = es[0]
            for v in es[1:]:
                tot = tot + v
            ws = [e / tot for e in es]
            for t in range(TOP_K):
                tok_idx = lane_tok + (2 * t)
                wv = _permute(ws[TOP_K - 1], tok_idx)
                iv = _permute(topi[TOP_K - 1], tok_idx)
                for j in range(TOP_K - 2, -1, -1):
                    wv = jnp.where(rank_masks[j], _permute(ws[j], tok_idx), wv)
                    iv = jnp.where(rank_masks[j], _permute(topi[j], tok_idx), iv)
                off = g * (L * TOP_K) + t * L
                ew_v[pl.ds(off, L)] = wv
                ei_v[pl.ds(off, L)] = iv
            return 0

        lax.fori_loop(0, ng, group, 0)
        pltpu.sync_copy(ew_v, ew_hbm.at[pl.ds(base * TOP_K, tok_w * TOP_K)])
        pltpu.sync_copy(ei_v, ei_hbm.at[pl.ds(base * TOP_K, tok_w * TOP_K)])

    return functools.partial(
        pl.kernel,
        out_type=[
            jax.ShapeDtypeStruct((tc * TOP_K,), jnp.float32),
            jax.ShapeDtypeStruct((tc * TOP_K,), jnp.int32),
        ],
        mesh=plsc.VectorSubcoreMesh(core_axis_name="c", subcore_axis_name="s"),
        scratch_types=[
            pltpu.VMEM((E, tok_w), jnp.float32),
            pltpu.VMEM((tok_w * TOP_K,), jnp.float32),
            pltpu.VMEM((tok_w * TOP_K,), jnp.int32),
            pltpu.SemaphoreType.DMA,
        ],
    )(_route_body)


_routes = {tc: _make_route(tc) for tc in set(CHUNK_TOKENS)}


@jax.jit
def kernel(x, attention_mask, W, b):
    x2 = x.reshape(T, D)
    m2 = attention_mask.reshape(1, T)
    b2 = b.reshape(E, 1)

    off = 0
    ews, eis = [], []
    for tc in CHUNK_TOKENS:
        nblk = tc // BT
        blk0 = off // BT
        logits = pl.pallas_call(
            _logits_body,
            grid=(nblk,),
            in_specs=[
                pl.BlockSpec((BT, D), lambda i, blk0=blk0: (blk0 + i, 0)),
                pl.BlockSpec((1, BT), lambda i, blk0=blk0: (0, blk0 + i)),
                pl.BlockSpec((E, D), lambda i: (0, 0)),
                pl.BlockSpec((E, 1), lambda i: (0, 0)),
            ],
            out_specs=pl.BlockSpec((E, BT), lambda i: (0, i)),
            out_shape=jax.ShapeDtypeStruct((E, tc), jnp.float32),
        )(x2, m2, W, b2)
        ew_c, ei_c = _routes[tc](logits)
        ews.append(ew_c)
        eis.append(ei_c)
        off += tc

    ew = jnp.concatenate(ews)
    ei = jnp.concatenate(eis)
    return (
        ew.reshape(B, S, TOP_K),
        ei.reshape(B, S, TOP_K),
    )


# speedup vs baseline: 1.2677x; 1.2677x over previous
import functools

import jax
import jax.numpy as jnp
from jax import lax
from jax.experimental import pallas as pl
from jax.experimental.pallas import tpu as pltpu
from jax.experimental.pallas import tpu_sc as plsc

B, S, D, E, TOP_K = 4, 4096, 4096, 64, 8
T = B * S

BT = 1024
CHUNK_TOKENS = (12288, 4096)

NC, NS, L = 2, 16, 16
NW = NC * NS


def _logits_body(x_ref, m_ref, w_ref, b_ref, lg_ref):
    lg = lax.dot_general(
        w_ref[...], x_ref[...],
        dimension_numbers=(((1,), (1,)), ((), ())),
        preferred_element_type=jnp.float32,
    )
    lg = lg + b_ref[...]
    lg_ref[...] = jnp.where(m_ref[...] != 1, -jnp.inf, lg)


def _make_route(tc):
    tok_w = tc // NW
    ng = tok_w // L

    def _route_body(lg_hbm, ew_hbm, ei_hbm, lg_v, ew_v, ei_v, sem):
        wid = lax.axis_index("s") * NC + lax.axis_index("c")
        base = wid * tok_w
        pltpu.sync_copy(lg_hbm.at[:, pl.ds(base, tok_w)], lg_v)

        neg_inf = jnp.full((L,), -jnp.inf, jnp.float32)

        def group(g, _):
            def expert(e, carry):
                topv = list(carry[:TOP_K])
                topi = list(carry[TOP_K:])
                xv = lg_v[e, pl.ds(g * L, L)]
                xi = jnp.full((L,), e, jnp.int32)
                for j in range(TOP_K):
                    c = xv > topv[j]
                    nv = jnp.where(c, xv, topv[j])
                    xv = jnp.where(c, topv[j], xv)
                    ni = jnp.where(c, xi, topi[j])
                    xi = jnp.where(c, topi[j], xi)
                    topv[j] = nv
                    topi[j] = ni
                return tuple(topv) + tuple(topi)

            init = (neg_inf,) * TOP_K + (jnp.zeros((L,), jnp.int32),) * TOP_K
            carry = lax.fori_loop(0, E, expert, init)
            topv = list(carry[:TOP_K])
            topi = list(carry[TOP_K:])
            es = [jnp.exp(v - topv[0]) for v in topv]
            tot = es[0]
            for v in es[1:]:
                tot = tot + v
            for j in range(TOP_K):
                ew_v[j, pl.ds(g * L, L)] = es[j] / tot
                ei_v[j, pl.ds(g * L, L)] = topi[j]
            return 0

        lax.fori_loop(0, ng, group, 0)
        pltpu.sync_copy(ew_v, ew_hbm.at[:, pl.ds(base, tok_w)])
        pltpu.sync_copy(ei_v, ei_hbm.at[:, pl.ds(base, tok_w)])

    return functools.partial(
        pl.kernel,
        out_type=[
            jax.ShapeDtypeStruct((TOP_K, tc), jnp.float32),
            jax.ShapeDtypeStruct((TOP_K, tc), jnp.int32),
        ],
        mesh=plsc.VectorSubcoreMesh(core_axis_name="c", subcore_axis_name="s"),
        scratch_types=[
            pltpu.VMEM((E, tok_w), jnp.float32),
            pltpu.VMEM((TOP_K, tok_w), jnp.float32),
            pltpu.VMEM((TOP_K, tok_w), jnp.int32),
            pltpu.SemaphoreType.DMA,
        ],
    )(_route_body)


_routes = {tc: _make_route(tc) for tc in set(CHUNK_TOKENS)}


@jax.jit
def kernel(x, attention_mask, W, b):
    x2 = x.reshape(T, D)
    m2 = attention_mask.reshape(1, T)
    b2 = b.reshape(E, 1)

    off = 0
    ews, eis = [], []
    for tc in CHUNK_TOKENS:
        nblk = tc // BT
        blk0 = off // BT
        logits = pl.pallas_call(
            _logits_body,
            grid=(nblk,),
            in_specs=[
                pl.BlockSpec((BT, D), lambda i, blk0=blk0: (blk0 + i, 0)),
                pl.BlockSpec((1, BT), lambda i, blk0=blk0: (0, blk0 + i)),
                pl.BlockSpec((E, D), lambda i: (0, 0)),
                pl.BlockSpec((E, 1), lambda i: (0, 0)),
            ],
            out_specs=pl.BlockSpec((E, BT), lambda i: (0, i)),
            out_shape=jax.ShapeDtypeStruct((E, tc), jnp.float32),
        )(x2, m2, W, b2)
        ew_c, ei_c = _routes[tc](logits)
        ews.append(ew_c)
        eis.append(ei_c)
        off += tc

    ew = jnp.concatenate(ews, axis=1)
    ei = jnp.concatenate(eis, axis=1)
    return (
        ew.T.reshape(B, S, TOP_K),
        ei.T.reshape(B, S, TOP_K),
    )
